# baseline (speedup 1.0000x reference)
import functools

import jax
import jax.numpy as jnp
from jax import lax
from jax.experimental import pallas as pl
from jax.experimental.pallas import tpu as pltpu
from jax.experimental.pallas import tpu_sc as plsc

BATCH = 16384
D = 64
NS = 16
BPW = BATCH // NS
PASS = 512
CH = 128
NCH = PASS // CH
L = 16


def _cf_body(uid_hbm, mid_hbm, ut_hbm, mt_hbm, wb_hbm, bb_hbm, out_hbm,
             uidx, midx, urows, mrows, wv, bv, outv, usem, msem):
    wid = lax.axis_index("s")
    iota16 = lax.iota(jnp.int32, L)
    bvec_init = None

    for p in range(BPW // PASS):
        base = wid * BPW + p * PASS

        pltpu.sync_copy(uid_hbm.at[pl.ds(base, PASS)], uidx)
        pltpu.sync_copy(mid_hbm.at[pl.ds(base, PASS)], midx)
        if p == 0:
            pltpu.sync_copy(wb_hbm, wv)
            pltpu.sync_copy(bb_hbm, bv)

        ucopies = [
            pltpu.async_copy(
                ut_hbm.at[uidx.at[pl.ds(j * CH, CH)]],
                urows.at[pl.ds(j * CH, CH)], usem)
            for j in range(NCH)
        ]
        mcopies = [
            pltpu.async_copy(
                mt_hbm.at[midx.at[pl.ds(j * CH, CH)]],
                mrows.at[pl.ds(j * CH, CH)], msem)
            for j in range(NCH)
        ]
        for c in ucopies:
            c.wait()
        for c in mcopies:
            c.wait()

        bvec = bv[...]

        def group(g, carry):
            ridx = g * L + iota16
            acc = bvec
            colv = jnp.zeros((L,), jnp.int32)
            for k in range(D):
                ucol = plsc.load_gather(urows, [ridx, colv])
                mcol = plsc.load_gather(mrows, [ridx, colv])
                acc = acc + ucol * wv[k]
                acc = acc + mcol * wv[D + k]
                if k + 1 < D:
                    colv = colv + 1
            outv[pl.ds(g * L, L)] = acc
            return carry

        lax.fori_loop(0, PASS // L, group, 0)

        pltpu.sync_copy(outv, out_hbm.at[pl.ds(base, PASS)])


@jax.jit
def _cf_call(uid, mid, user_table, movie_table, wb, bb):
    mesh = plsc.VectorSubcoreMesh(
        core_axis_name="c", subcore_axis_name="s", num_cores=1
    )
    f = functools.partial(
        pl.kernel,
        mesh=mesh,
        compiler_params=pltpu.CompilerParams(
            needs_layout_passes=False, use_tc_tiling_on_sc=False
        ),
        out_type=jax.ShapeDtypeStruct((BATCH,), jnp.float32),
        scratch_types=[
            pltpu.VMEM((PASS,), jnp.int32),
            pltpu.VMEM((PASS,), jnp.int32),
            pltpu.VMEM((PASS, D), jnp.float32),
            pltpu.VMEM((PASS, D), jnp.float32),
            pltpu.VMEM((2 * D, L), jnp.float32),
            pltpu.VMEM((L,), jnp.float32),
            pltpu.VMEM((PASS,), jnp.float32),
            pltpu.SemaphoreType.DMA,
            pltpu.SemaphoreType.DMA,
        ],
    )(_cf_body)
    return f(uid, mid, user_table, movie_table, wb, bb)


def kernel(user_ids, movie_ids, user_table, movie_table, W, b):
    wb = jnp.broadcast_to(W.reshape(2 * D, 1), (2 * D, L))
    bb = jnp.broadcast_to(b.reshape(1), (L,))
    return _cf_call(
        user_ids.astype(jnp.int32), movie_ids.astype(jnp.int32),
        user_table, movie_table, wb, bb,
    )


# speedup vs baseline: 1.6028x; 1.6028x over previous
import functools

import jax
import jax.numpy as jnp
from jax import lax
from jax.experimental import pallas as pl
from jax.experimental.pallas import tpu as pltpu
from jax.experimental.pallas import tpu_sc as plsc

BATCH = 16384
D = 64
NC = 2
NS = 16
NW = NC * NS
BPW = BATCH // NW
L = 16
BLK = 16
NBLK = BPW // BLK
NBUF = 2


def _cf_body(uid_hbm, mid_hbm, ut_hbm, mt_hbm, wb_hbm, out_hbm,
             uidx, midx, uring, mring, wv, outv, usem, msem):
    wid = lax.axis_index("s") * NC + lax.axis_index("c")
    base = wid * BPW

    pltpu.sync_copy(uid_hbm.at[pl.ds(base, BPW)], uidx)
    pltpu.sync_copy(mid_hbm.at[pl.ds(base, BPW)], midx)
    pltpu.sync_copy(wb_hbm, wv)

    lane = lax.iota(jnp.int32, L)
    last_lane = lane == (L - 1)

    def issue_block(blk, par):
        uids = uidx[pl.ds(blk * BLK, L)] & ~7
        mids = midx[pl.ds(blk * BLK, L)] & ~7
        for r in range(BLK):
            slot = par * BLK + r
            pltpu.async_copy(
                ut_hbm.at[pl.ds(pl.multiple_of(uids[r], 8), 8), :],
                uring.at[slot], usem)
            pltpu.async_copy(
                mt_hbm.at[pl.ds(pl.multiple_of(mids[r], 8), 8), :],
                mring.at[slot], msem)

    def wait_block(par):
        for r in range(BLK):
            slot = par * BLK + r
            pltpu.make_async_copy(
                ut_hbm.at[pl.ds(0, 8), :], uring.at[slot], usem).wait()
            pltpu.make_async_copy(
                mt_hbm.at[pl.ds(0, 8), :], mring.at[slot], msem).wait()

    issue_block(0, 0)

    w = [wv[pl.ds(k * L, L)] for k in range(2 * D // L)]
    bvec = wv[pl.ds(2 * D, L)]

    def block(g, carry):
        par = lax.rem(g, NBUF)
        wait_block(par)

        @pl.when(g + 1 < NBLK)
        def _():
            issue_block(g + 1, lax.rem(g + 1, NBUF))

        row0 = g * BLK
        usub = uidx[pl.ds(row0, L)] & 7
        msub = midx[pl.ds(row0, L)] & 7
        for r in range(BLK):
            slot = par * BLK + r
            ur = usub[r]
            mr = msub[r]
            acc = None
            for k in range(D // L):
                uv = uring[slot, ur, pl.ds(k * L, L)] * w[k]
                mv = mring[slot, mr, pl.ds(k * L, L)] * w[D // L + k]
                t = uv + mv
                acc = t if acc is None else acc + t
            s = plsc.cumsum(acc) + bvec
            plsc.store_scatter(
                outv, [jnp.full((L,), row0 + r, jnp.int32)], s, mask=last_lane
            )
        return carry

    lax.fori_loop(0, NBLK, block, 0)

    pltpu.sync_copy(outv, out_hbm.at[pl.ds(base, BPW)])


@jax.jit
def _cf_call(user_ids, movie_ids, user_table, movie_table, wb):
    mesh = plsc.VectorSubcoreMesh(core_axis_name="c", subcore_axis_name="s")
    f = functools.partial(
        pl.kernel,
        mesh=mesh,
        compiler_params=pltpu.CompilerParams(needs_layout_passes=False),
        out_type=jax.ShapeDtypeStruct((BATCH,), jnp.float32),
        scratch_types=[
            pltpu.VMEM((BPW,), jnp.int32),
            pltpu.VMEM((BPW,), jnp.int32),
            pltpu.VMEM((NBUF * BLK, 8, D), jnp.float32),
            pltpu.VMEM((NBUF * BLK, 8, D), jnp.float32),
            pltpu.VMEM((2 * D + L,), jnp.float32),
            pltpu.VMEM((BPW,), jnp.float32),
            pltpu.SemaphoreType.DMA,
            pltpu.SemaphoreType.DMA,
        ],
    )(_cf_body)
    return f(user_ids, movie_ids, user_table, movie_table, wb)


def kernel(user_ids, movie_ids, user_table, movie_table, W, b):
    wb = jnp.concatenate(
        [W.reshape(2 * D), jnp.broadcast_to(b.reshape(1), (L,))]
    )
    return _cf_call(
        user_ids.astype(jnp.int32), movie_ids.astype(jnp.int32),
        user_table, movie_table, wb,
    )
